# baseline (speedup 1.0000x reference)
import functools

import jax
import jax.numpy as jnp
from jax import lax
from jax.experimental import pallas as pl
from jax.experimental.pallas import tpu as pltpu
from jax.experimental.pallas import tpu_sc as plsc

_NL = 4
_NP = 2
_NG = 64
_N = 50000
_B = 10000
_D = 96
_S = 32
_F = 128
_RB = 1000
_EP_SMALL = 163840
_EP_BIG = 819200

_f32 = jnp.float32


def _stpre(st, wi, wf, wb, w1):
    def body(st_ref, wi_ref, wf_ref, wb_ref, w1_ref, oi, of, ob, o1):
        s = st_ref[...]
        oi[...] = jnp.dot(s, wi_ref[...], preferred_element_type=_f32)
        of[...] = jnp.dot(s, wf_ref[...], preferred_element_type=_f32)
        ob[...] = jnp.dot(s, wb_ref[...], preferred_element_type=_f32)
        o1[...] = jnp.dot(jnp.maximum(s, 0.0), w1_ref[...],
                          preferred_element_type=_f32)
    wspec = pl.BlockSpec((_S, _D), lambda i: (0, 0))
    return pl.pallas_call(
        body,
        grid=(_N // _RB,),
        in_specs=[pl.BlockSpec((_RB, _S), lambda i: (i, 0)),
                  wspec, wspec, wspec,
                  pl.BlockSpec((_S, _F), lambda i: (0, 0))],
        out_specs=[pl.BlockSpec((_RB, _D), lambda i: (i, 0))] * 3
                  + [pl.BlockSpec((_RB, _F), lambda i: (i, 0))],
        out_shape=[jax.ShapeDtypeStruct((_N, _D), _f32)] * 3
                  + [jax.ShapeDtypeStruct((_N, _F), _f32)],
    )(st, wi, wf, wb, w1)


def _pre(h, stp, boff, deg, w):
    n = h.shape[0]
    def body(h_ref, stp_ref, deg_ref, w_ref, o0, o1):
        dv = deg_ref[...]
        dinv = lax.rsqrt(dv[0] + dv[1] + 1.0)
        g = (jnp.dot(h_ref[...], w_ref[...], preferred_element_type=_f32)
             + stp_ref[...]) * dinv
        o0[...] = g[:, 0:48]
        o1[...] = g[:, 48:96]
    return pl.pallas_call(
        body,
        grid=(n // _RB,),
        in_specs=[pl.BlockSpec((_RB, _D), lambda i: (i, 0)),
                  pl.BlockSpec((_RB, _D), lambda i: (i + boff, 0)),
                  pl.BlockSpec((2, _RB, 1), lambda i: (0, i, 0)),
                  pl.BlockSpec((_D, _D), lambda i: (0, 0))],
        out_specs=[pl.BlockSpec((_RB, 48), lambda i: (i, 0))] * 2,
        out_shape=[jax.ShapeDtypeStruct((2 * _B, 48), _f32)] * 2,
    )(h, stp, deg, w)


def _combine(a0, a1, g0, g1, deg, b, ioff, nout):
    def body(a0_ref, a1_ref, g0_ref, g1_ref, deg_ref, b_ref, o_ref):
        av0 = a0_ref[...]
        av1 = a1_ref[...]
        m = jnp.concatenate([av0[0] + av1[0] + g0_ref[...],
                             av0[1] + av1[1] + g1_ref[...]], axis=1)
        dv = deg_ref[...]
        o_ref[...] = m * lax.rsqrt(dv[0] + dv[1] + 1.0) + b_ref[...]
    aspec = pl.BlockSpec((2, _RB, 48), lambda i: (0, i + ioff, 0))
    gspec = pl.BlockSpec((_RB, 48), lambda i: (i + ioff, 0))
    return pl.pallas_call(
        body,
        grid=(nout // _RB,),
        in_specs=[aspec, aspec, gspec, gspec,
                  pl.BlockSpec((2, _RB, 1), lambda i: (0, i + ioff, 0)),
                  pl.BlockSpec((1, _D), lambda i: (0, 0))],
        out_specs=pl.BlockSpec((_RB, _D), lambda i: (i, 0)),
        out_shape=jax.ShapeDtypeStruct((nout, _D), _f32),
    )(a0, a1, g0, g1, deg, b)


def _combine_dnn(a0, a1, g0, g1, deg, bg, ioff, sr1, boff, w1d, w2, w3,
                 b1, b2, b3):
    def body(a0r, a1r, g0r, g1r, deg_ref, bg_ref, s_ref, w1r, w2r, w3r,
             b1r, b2r, b3r, o_ref):
        av0 = a0r[...]
        av1 = a1r[...]
        m = jnp.concatenate([av0[0] + av1[0] + g0r[...],
                             av0[1] + av1[1] + g1r[...]], axis=1)
        dv = deg_ref[...]
        h = m * lax.rsqrt(dv[0] + dv[1] + 1.0) + bg_ref[...]
        t = jnp.maximum(h, 0.0)
        t = jnp.maximum(jnp.dot(t, w1r[...], preferred_element_type=_f32)
                        + s_ref[...] + b1r[...], 0.0)
        t = jnp.maximum(jnp.dot(t, w2r[...], preferred_element_type=_f32)
                        + b2r[...], 0.0)
        o_ref[...] = jnp.maximum(jnp.dot(t, w3r[...],
                                         preferred_element_type=_f32)
                                 + b3r[...], 0.0)
    aspec = pl.BlockSpec((2, _RB, 48), lambda i: (0, i + ioff, 0))
    gspec = pl.BlockSpec((_RB, 48), lambda i: (i + ioff, 0))
    return pl.pallas_call(
        body,
        grid=(_B // _RB,),
        in_specs=[aspec, aspec, gspec, gspec,
                  pl.BlockSpec((2, _RB, 1), lambda i: (0, i + ioff, 0)),
                  pl.BlockSpec((1, _D), lambda i: (0, 0)),
                  pl.BlockSpec((_RB, _F), lambda i: (i + boff, 0)),
                  pl.BlockSpec((_D, _F), lambda i: (0, 0)),
                  pl.BlockSpec((_F, _F), lambda i: (0, 0)),
                  pl.BlockSpec((_F, _D), lambda i: (0, 0)),
                  pl.BlockSpec((1, _F), lambda i: (0, 0)),
                  pl.BlockSpec((1, _F), lambda i: (0, 0)),
                  pl.BlockSpec((1, _D), lambda i: (0, 0))],
        out_specs=pl.BlockSpec((_RB, _D), lambda i: (i, 0)),
        out_shape=jax.ShapeDtypeStruct((_B, _D), _f32),
    )(a0, a1, g0, g1, deg, bg, sr1, w1d, w2, w3, b1, b2, b3)


def _combine_pre(a0, a1, g0, g1, degc, bg, stp, boff, degn, w):
    def body(a0r, a1r, g0r, g1r, dc_ref, bg_ref, stp_ref, dn_ref, w_ref,
             o0, o1):
        av0 = a0r[...]
        av1 = a1r[...]
        m = jnp.concatenate([av0[0] + av1[0] + g0r[...],
                             av0[1] + av1[1] + g1r[...]], axis=1)
        dv = dc_ref[...]
        h = m * lax.rsqrt(dv[0] + dv[1] + 1.0) + bg_ref[...]
        dn = dn_ref[...]
        g = (jnp.dot(h, w_ref[...], preferred_element_type=_f32)
             + stp_ref[...]) * lax.rsqrt(dn[0] + dn[1] + 1.0)
        o0[...] = g[:, 0:48]
        o1[...] = g[:, 48:96]
    aspec = pl.BlockSpec((2, _RB, 48), lambda i: (0, i, 0))
    gspec = pl.BlockSpec((_RB, 48), lambda i: (i, 0))
    return pl.pallas_call(
        body,
        grid=(_B // _RB,),
        in_specs=[aspec, aspec, gspec, gspec,
                  pl.BlockSpec((2, _RB, 1), lambda i: (0, i, 0)),
                  pl.BlockSpec((1, _D), lambda i: (0, 0)),
                  pl.BlockSpec((_RB, _D), lambda i: (i + boff, 0)),
                  pl.BlockSpec((2, _RB, 1), lambda i: (0, i, 0)),
                  pl.BlockSpec((_D, _D), lambda i: (0, 0))],
        out_specs=[pl.BlockSpec((_RB, 48), lambda i: (i, 0))] * 2,
        out_shape=[jax.ShapeDtypeStruct((2 * _B, 48), _f32)] * 2,
    )(a0, a1, g0, g1, degc, bg, stp, degn, w)


def _pre_big(x, wup, deg):
    def body(x_ref, w_ref, deg_ref, *outs):
        dv = deg_ref[...]
        dinv = lax.rsqrt(dv[0] + dv[1] + 1.0)
        g = jnp.dot(x_ref[...], w_ref[...], preferred_element_type=_f32) * dinv
        for k in range(6):
            outs[k][...] = g[:, 16 * k:16 * (k + 1)]
    return pl.pallas_call(
        body,
        grid=(_N // _RB,),
        in_specs=[pl.BlockSpec((_RB, _S), lambda i: (i, 0)),
                  pl.BlockSpec((_S, _D), lambda i: (0, 0)),
                  pl.BlockSpec((2, _RB, 1), lambda i: (0, i, 0))],
        out_specs=[pl.BlockSpec((_RB, 16), lambda i: (i, 0))] * 6,
        out_shape=[jax.ShapeDtypeStruct((_N, 16), _f32)] * 6,
    )(x, wup, deg)


def _combine_big(a0, a1, gs, deg, b):
    def body(a0r, a1r, *rest):
        grs = rest[:6]
        deg_ref, b_ref, o_ref = rest[6:]
        av0 = a0r[...]
        av1 = a1r[...]
        m = jnp.concatenate(
            [av0[k] + av1[k] + grs[k][...] for k in range(6)], axis=1)
        dv = deg_ref[...]
        o_ref[...] = m * lax.rsqrt(dv[0] + dv[1] + 1.0) + b_ref[...]
    aspec = pl.BlockSpec((6, _RB, 16), lambda i: (0, i, 0))
    cspec = pl.BlockSpec((_RB, 16), lambda i: (i, 0))
    return pl.pallas_call(
        body,
        grid=(_N // _RB,),
        in_specs=[aspec, aspec] + [cspec] * 6
                 + [pl.BlockSpec((2, _RB, 1), lambda i: (0, i, 0)),
                    pl.BlockSpec((1, _D), lambda i: (0, 0))],
        out_specs=pl.BlockSpec((_RB, _D), lambda i: (i, 0)),
        out_shape=jax.ShapeDtypeStruct((_N, _D), _f32),
    )(a0, a1, *gs, deg, b)


def _dnn(h, sr1, boff, w1d, w2, w3, b1, b2, b3):
    def body(h_ref, s_ref, w1r, w2r, w3r, b1r, b2r, b3r, o_ref):
        t = jnp.maximum(h_ref[...], 0.0)
        t = jnp.maximum(jnp.dot(t, w1r[...], preferred_element_type=_f32)
                        + s_ref[...] + b1r[...], 0.0)
        t = jnp.maximum(jnp.dot(t, w2r[...], preferred_element_type=_f32)
                        + b2r[...], 0.0)
        o_ref[...] = jnp.maximum(jnp.dot(t, w3r[...],
                                         preferred_element_type=_f32)
                                 + b3r[...], 0.0)
    return pl.pallas_call(
        body,
        grid=(_B // _RB,),
        in_specs=[pl.BlockSpec((_RB, _D), lambda i: (i, 0)),
                  pl.BlockSpec((_RB, _F), lambda i: (i + boff, 0)),
                  pl.BlockSpec((_D, _F), lambda i: (0, 0)),
                  pl.BlockSpec((_F, _F), lambda i: (0, 0)),
                  pl.BlockSpec((_F, _D), lambda i: (0, 0)),
                  pl.BlockSpec((1, _F), lambda i: (0, 0)),
                  pl.BlockSpec((1, _F), lambda i: (0, 0)),
                  pl.BlockSpec((1, _D), lambda i: (0, 0))],
        out_specs=pl.BlockSpec((_RB, _D), lambda i: (i, 0)),
        out_shape=jax.ShapeDtypeStruct((_B, _D), _f32),
    )(h, sr1, w1d, w2, w3, b1, b2, b3)


def _pool(h, ids, wend, bend):
    def body(h_ref, id_ref, we_ref, be_ref, o_ref, acc_ref):
        i = pl.program_id(0)
        @pl.when(i == 0)
        def _():
            acc_ref[...] = jnp.zeros_like(acc_ref)
        oh = (id_ref[...] == lax.broadcasted_iota(jnp.int32, (_RB, _NG), 1)
              ).astype(_f32)
        acc_ref[...] += lax.dot_general(oh, h_ref[...],
                                        (((0,), (0,)), ((), ())),
                                        preferred_element_type=_f32)
        @pl.when(i == pl.num_programs(0) - 1)
        def _():
            o_ref[...] = jnp.maximum(
                jnp.dot(acc_ref[...], we_ref[...],
                        preferred_element_type=_f32) + be_ref[...], 0.0)
    return pl.pallas_call(
        body,
        grid=(_N // _RB,),
        in_specs=[pl.BlockSpec((_RB, _D), lambda i: (i, 0)),
                  pl.BlockSpec((_RB, 1), lambda i: (i, 0)),
                  pl.BlockSpec((_D, 1), lambda i: (0, 0)),
                  pl.BlockSpec((1, 1), lambda i: (0, 0))],
        out_specs=pl.BlockSpec((_NG, 1), lambda i: (0, 0)),
        out_shape=jax.ShapeDtypeStruct((_NG, 1), _f32),
        scratch_shapes=[pltpu.VMEM((_NG, _D), _f32)],
    )(h, ids, wend, bend)


def _sc_mesh():
    return plsc.VectorSubcoreMesh(core_axis_name="c", subcore_axis_name="s",
                                  num_cores=2, num_subcores=16)


@functools.lru_cache(maxsize=None)
def _mk_edgepass(n, ch, w, rows):
    npad = -(-(n + 16) // 128) * 128
    rps = npad // 16
    nread = (n // 16 // 8) * 8
    ntail = n - 16 * nread
    zchunks = []
    off = 0
    while off < rps:
        sz = min(128, rps - off)
        zchunks.append((off, sz))
        off += sz

    k = 4
    ngroups = rows // k
    assert rows % (2 * k) == 0

    def body(*refs):
        gs = refs[:ch]
        srcr, dstr, zr = refs[ch:ch + 3]
        outs = refs[ch + 3:ch + 5]
        sbuf, dbuf, rbufs, zbuf = refs[ch + 5:ch + 9]
        acc = refs[ch + 9]
        semg = refs[ch + 10:ch + 12]
        sems = refs[ch + 12:ch + 14]
        semz = refs[ch + 14]
        c = lax.axis_index("c")
        s = lax.axis_index("s")
        wid = c * 16 + s
        pltpu.sync_copy(srcr.at[pl.ds(wid * rows, rows)], sbuf)
        pltpu.sync_copy(dstr.at[pl.ds(wid * rows, rows)], dbuf)
        pltpu.sync_copy(zr, zbuf)

        def issue_gathers(gref, group, si):
            for b in range(k):
                pltpu.async_copy(gref.at[sbuf.at[group * k + b]],
                                 rbufs.at[si, b], semg[si])

        def drain_gathers(si):
            for b in range(k):
                pltpu.make_async_copy(zr, rbufs.at[si, b], semg[si]).wait()

        for chi in range(ch):
            zds = [pltpu.async_copy(zbuf.at[pl.ds(0, zsz)],
                                    acc.at[pl.ds(s * rps + zo, zsz)], semz)
                   for zo, zsz in zchunks]
            for d in zds:
                d.wait()
            plsc.subcore_barrier()
            issue_gathers(gs[chi], 0, 0)
            issue_gathers(gs[chi], 1, 1)

            def pair(p, carry, chi=chi):
                for si in (0, 1):
                    g = 2 * p + si
                    drain_gathers(si)
                    sds = [pltpu.async_copy(
                        rbufs.at[si, b], acc.at[dbuf.at[g * k + b]],
                        sems[si], add=True) for b in range(k)]
                    for d in sds:
                        d.wait()

                    @pl.when(g + 2 < ngroups)
                    def _(si=si, g=g):
                        issue_gathers(gs[chi], g + 2, si)
                return carry
            lax.fori_loop(0, ngroups // 2, pair, 0)
            plsc.subcore_barrier()
            for ci in range(2):
                @pl.when(c == ci)
                def _(chi=chi, ci=ci):
                    pltpu.sync_copy(acc.at[pl.ds(s * nread, nread)],
                                    outs[ci].at[chi, pl.ds(s * nread,
                                                           nread)])
                    if ntail:
                        @pl.when(s == 0)
                        def _():
                            pltpu.sync_copy(
                                acc.at[pl.ds(16 * nread, ntail)],
                                outs[ci].at[chi, pl.ds(16 * nread, ntail)])
            plsc.subcore_barrier()

    return pl.kernel(
        body,
        out_type=(jax.ShapeDtypeStruct((ch, n, w), _f32),
                  jax.ShapeDtypeStruct((ch, n, w), _f32)),
        mesh=_sc_mesh(),
        compiler_params=pltpu.CompilerParams(use_tc_tiling_on_sc=False),
        scratch_types=[pltpu.VMEM((rows, 128), jnp.int32),
                       pltpu.VMEM((rows, 128), jnp.int32),
                       pltpu.VMEM((2, k, 128, w), _f32),
                       pltpu.VMEM((128, w), _f32),
                       pltpu.VMEM_SHARED((npad, w), _f32),
                       pltpu.SemaphoreType.DMA,
                       pltpu.SemaphoreType.DMA,
                       pltpu.SemaphoreType.DMA,
                       pltpu.SemaphoreType.DMA,
                       pltpu.SemaphoreType.DMA])


@functools.lru_cache(maxsize=None)
def _mk_degree(slots, rows):
    sps = slots // 16
    k = 10
    ngroups = rows // k
    assert rows % (2 * k) == 0

    def body(dstrA, dstrB, zeros_r, oA0, oA1, oB0, oB1, dbuf, obuf, stage,
             acc, sem0, sem1):
        c = lax.axis_index("c")
        s = lax.axis_index("s")
        wid = c * 16 + s
        sems = (sem0, sem1)
        for i in range(8):
            obuf[pl.ds(i * 16, 16)] = jnp.ones((16,), _f32)
        pltpu.sync_copy(zeros_r, stage)

        def issue(group, si):
            for b in range(k):
                pltpu.async_copy(obuf, acc.at[dbuf.at[group * k + b]],
                                 sems[si], add=True)

        def drain(si):
            for b in range(k):
                pltpu.make_async_copy(zeros_r.at[pl.ds(0, 128)], obuf,
                                      sems[si]).wait()

        for dstr, outs in ((dstrA, (oA0, oA1)), (dstrB, (oB0, oB1))):
            pltpu.sync_copy(dstr.at[pl.ds(wid * rows, rows)], dbuf)
            pltpu.sync_copy(stage, acc.at[pl.ds(s * sps, sps)])
            plsc.subcore_barrier()
            issue(0, 0)

            def pair(p, carry):
                issue(2 * p + 1, 1)
                drain(0)

                @pl.when(2 * p + 2 < ngroups)
                def _():
                    issue(2 * p + 2, 0)
                drain(1)
                return carry
            lax.fori_loop(0, ngroups // 2, pair, 0)
            plsc.subcore_barrier()
            pltpu.sync_copy(acc.at[pl.ds(s * sps, sps)], stage)
            for ci, outr in enumerate(outs):
                @pl.when(c == ci)
                def _(outr=outr):
                    pltpu.sync_copy(stage, outr.at[pl.ds(s * sps, sps)])
            plsc.subcore_barrier()
            pltpu.sync_copy(zeros_r, stage)

    return pl.kernel(
        body,
        out_type=tuple(jax.ShapeDtypeStruct((slots,), _f32)
                       for _ in range(4)),
        mesh=_sc_mesh(),
        compiler_params=pltpu.CompilerParams(use_tc_tiling_on_sc=False),
        scratch_types=[pltpu.VMEM((rows, 128), jnp.int32),
                       pltpu.VMEM((128,), _f32),
                       pltpu.VMEM((sps,), _f32),
                       pltpu.VMEM_SHARED((slots,), _f32),
                       pltpu.SemaphoreType.DMA,
                       pltpu.SemaphoreType.DMA])


def _pad_edges(src, dst, n, epad):
    e = src.shape[0]
    srcp = jnp.concatenate([src, jnp.zeros((epad - e,), jnp.int32)])
    dstp = jnp.concatenate([dst, jnp.full((epad - e,), n, jnp.int32)])
    return srcp.reshape(epad // 128, 128), dstp.reshape(epad // 128, 128), dstp


def kernel(x, feature_mtx_static, edge_index, inner_edges, forward_edges,
           backward_edges, batch_ids, W_up, b_up, W_in, b_in, W_f, b_f,
           W_b, b_b, W1, b1, W2, b2, W3, b3, W_end, b_end):
    st = feature_mtx_static

    stp_i, stp_f, stp_b, sr1 = _stpre(st, W_in[_D:], W_f[_D:], W_b[_D:],
                                      W1[_D:])
    w_in_d, w_f_d, w_b_d, w1_d = W_in[:_D], W_f[:_D], W_b[:_D], W1[:_D]
    b_in2 = b_in.reshape(1, _D)
    b_f2 = b_f.reshape(1, _D)
    b_b2 = b_b.reshape(1, _D)
    b_up2 = b_up.reshape(1, _D)

    biglist = (edge_index[0], edge_index[1], _N, _EP_BIG, _N)
    inn = [(inner_edges[l, 0], inner_edges[l, 1], _B, _EP_SMALL, _B)
           for l in range(_NL)]
    fwd = [(forward_edges[l, 0], forward_edges[l, 1], 2 * _B, _EP_SMALL,
            2 * _B) for l in range(_NL - 1)]
    bwd = [(backward_edges[l, 0], backward_edges[l, 1], 2 * _B, _EP_SMALL,
            2 * _B) for l in range(1, _NL)]

    def prep(lst):
        return [_pad_edges(s_, d_, pd_, ep_)
                for (s_, d_, _n, ep_, pd_) in lst]

    bigP = prep([biglist])[0]
    innP = prep(inn)
    fwdP = prep(fwd)
    bwdP = prep(bwd)

    slots = 140160
    zeros_deg = jnp.zeros((slots // 16,), _f32)
    degfn = _mk_degree(slots, 10240 // 32)
    offsA = [0, 50016, 60032]
    dstA = jnp.concatenate(
        [bigP[2], innP[0][2] + offsA[1], innP[1][2] + offsA[2],
         jnp.full((1310720 - 1146880,), _N, jnp.int32)]).reshape(-1, 128)
    offsB = [0, 10016, 20032, 40048, 60064, 80080, 100096, 120112]
    dstB = jnp.concatenate(
        [innP[2][2], innP[3][2] + offsB[1],
         fwdP[0][2] + offsB[2], fwdP[1][2] + offsB[3],
         fwdP[2][2] + offsB[4],
         bwdP[0][2] + offsB[5], bwdP[1][2] + offsB[6],
         bwdP[2][2] + offsB[7]]).reshape(-1, 128)
    dA0, dA1, dB0, dB1 = degfn(dstA, dstB, zeros_deg)
    degA = jnp.stack([dA0, dA1])
    degB = jnp.stack([dB0, dB1])

    def deg_of(src, off, n_):
        return lax.slice(src, (0, off), (2, off + n_)).reshape(2, n_, 1)

    deg_big = deg_of(degA, 0, _N)
    deg_in = [deg_of(degA, offsA[1], _B), deg_of(degA, offsA[2], _B),
              deg_of(degB, offsB[0], _B), deg_of(degB, offsB[1], _B)]
    deg_fw = [deg_of(degB, offsB[2 + l], 2 * _B) for l in range(_NL - 1)]
    deg_bw = {l: deg_of(degB, offsB[5 + l - 1], 2 * _B)
              for l in range(1, _NL)}

    z48 = jnp.zeros((128, 48), _f32)
    z16 = jnp.zeros((128, 16), _f32)
    ep2 = _mk_edgepass(2 * _B, 2, 48, _EP_SMALL // 128 // 32)
    epbig = _mk_edgepass(_N, 6, 16, _EP_BIG // 128 // 32)

    g6 = _pre_big(x, W_up, deg_big)
    pa0, pa1 = epbig(*g6, bigP[0], bigP[1], z16)
    h_full = _combine_big(pa0, pa1, g6, deg_big, b_up2)
    hb = [lax.slice(h_full, (k * _B, 0), ((k + 1) * _B, _D))
          for k in range(_NL + 1)]

    def gcn_inner(l):
        g0, g1 = _pre(hb[l], stp_i, l * (_B // _RB), deg_in[l], w_in_d)
        a0, a1 = ep2(g0, g1, innP[l][0], innP[l][1], z48)
        hb[l] = _combine(a0, a1, g0, g1, deg_in[l], b_in2, 0, _B)

    for _ in range(_NP):
        for l in range(_NL):
            gcn_inner(l)
            if l == _NL - 1:
                continue
            hcat = jnp.concatenate([hb[l], hb[l + 1]])
            g0, g1 = _pre(hcat, stp_f, l * (_B // _RB), deg_fw[l], w_f_d)
            a0, a1 = ep2(g0, g1, fwdP[l][0], fwdP[l][1], z48)
            hb[l + 1] = _combine(a0, a1, g0, g1, deg_fw[l], b_f2,
                                 _B // _RB, _B)
            hb[l + 1] = _dnn(hb[l + 1], sr1, (l + 1) * (_B // _RB),
                             w1_d, W2, W3, b1.reshape(1, _F),
                             b2.reshape(1, _F), b3.reshape(1, _D))
        for l in range(_NL - 1, 0, -1):
            hcat = jnp.concatenate([hb[l - 1], hb[l]])
            g0, g1 = _pre(hcat, stp_b, (l - 1) * (_B // _RB), deg_bw[l],
                          w_b_d)
            a0, a1 = ep2(g0, g1, bwdP[l - 1][0], bwdP[l - 1][1], z48)
            hb[l - 1] = _combine(a0, a1, g0, g1, deg_bw[l], b_b2, 0, _B)
            gcn_inner(l - 1)
            hb[l - 1] = _dnn(hb[l - 1], sr1, (l - 1) * (_B // _RB),
                             w1_d, W2, W3, b1.reshape(1, _F),
                             b2.reshape(1, _F), b3.reshape(1, _D))

    h_all = jnp.concatenate(hb)
    return _pool(h_all, batch_ids.reshape(_N, 1), W_end, b_end.reshape(1, 1))


# speedup vs baseline: 1.0577x; 1.0577x over previous
import functools

import jax
import jax.numpy as jnp
from jax import lax
from jax.experimental import pallas as pl
from jax.experimental.pallas import tpu as pltpu
from jax.experimental.pallas import tpu_sc as plsc

_NL = 4
_NP = 2
_NG = 64
_N = 50000
_B = 10000
_D = 96
_S = 32
_F = 128
_RB = 1000
_EP_SMALL = 163840
_EP_BIG = 819200

_f32 = jnp.float32


def _stpre(st, wi, wf, wb, w1):
    def body(st_ref, wi_ref, wf_ref, wb_ref, w1_ref, oi, of, ob, o1):
        s = st_ref[...]
        oi[...] = jnp.dot(s, wi_ref[...], preferred_element_type=_f32)
        of[...] = jnp.dot(s, wf_ref[...], preferred_element_type=_f32)
        ob[...] = jnp.dot(s, wb_ref[...], preferred_element_type=_f32)
        o1[...] = jnp.dot(jnp.maximum(s, 0.0), w1_ref[...],
                          preferred_element_type=_f32)
    wspec = pl.BlockSpec((_S, _D), lambda i: (0, 0))
    return pl.pallas_call(
        body,
        grid=(_N // _RB,),
        in_specs=[pl.BlockSpec((_RB, _S), lambda i: (i, 0)),
                  wspec, wspec, wspec,
                  pl.BlockSpec((_S, _F), lambda i: (0, 0))],
        out_specs=[pl.BlockSpec((_RB, _D), lambda i: (i, 0))] * 3
                  + [pl.BlockSpec((_RB, _F), lambda i: (i, 0))],
        out_shape=[jax.ShapeDtypeStruct((_N, _D), _f32)] * 3
                  + [jax.ShapeDtypeStruct((_N, _F), _f32)],
    )(st, wi, wf, wb, w1)


def _pre(h, stp, boff, deg, w):
    n = h.shape[0]
    def body(h_ref, stp_ref, deg_ref, w_ref, o0, o1):
        dv = deg_ref[...]
        dinv = lax.rsqrt(dv[0] + dv[1] + 1.0)
        g = (jnp.dot(h_ref[...], w_ref[...], preferred_element_type=_f32)
             + stp_ref[...]) * dinv
        o0[...] = g[:, 0:48]
        o1[...] = g[:, 48:96]
    return pl.pallas_call(
        body,
        grid=(n // _RB,),
        in_specs=[pl.BlockSpec((_RB, _D), lambda i: (i, 0)),
                  pl.BlockSpec((_RB, _D), lambda i: (i + boff, 0)),
                  pl.BlockSpec((2, _RB, 1), lambda i: (0, i, 0)),
                  pl.BlockSpec((_D, _D), lambda i: (0, 0))],
        out_specs=[pl.BlockSpec((_RB, 48), lambda i: (i, 0))] * 2,
        out_shape=[jax.ShapeDtypeStruct((2 * _B, 48), _f32)] * 2,
    )(h, stp, deg, w)


def _combine(a0, a1, g0, g1, deg, b, ioff, nout):
    def body(a0_ref, a1_ref, g0_ref, g1_ref, deg_ref, b_ref, o_ref):
        av0 = a0_ref[...]
        av1 = a1_ref[...]
        m = jnp.concatenate([av0[0] + av1[0] + g0_ref[...],
                             av0[1] + av1[1] + g1_ref[...]], axis=1)
        dv = deg_ref[...]
        o_ref[...] = m * lax.rsqrt(dv[0] + dv[1] + 1.0) + b_ref[...]
    aspec = pl.BlockSpec((2, _RB, 48), lambda i: (0, i + ioff, 0))
    gspec = pl.BlockSpec((_RB, 48), lambda i: (i + ioff, 0))
    return pl.pallas_call(
        body,
        grid=(nout // _RB,),
        in_specs=[aspec, aspec, gspec, gspec,
                  pl.BlockSpec((2, _RB, 1), lambda i: (0, i + ioff, 0)),
                  pl.BlockSpec((1, _D), lambda i: (0, 0))],
        out_specs=pl.BlockSpec((_RB, _D), lambda i: (i, 0)),
        out_shape=jax.ShapeDtypeStruct((nout, _D), _f32),
    )(a0, a1, g0, g1, deg, b)


def _combine_dnn(a0, a1, g0, g1, deg, bg, ioff, sr1, boff, w1d, w2, w3,
                 b1, b2, b3):
    def body(a0r, a1r, g0r, g1r, deg_ref, bg_ref, s_ref, w1r, w2r, w3r,
             b1r, b2r, b3r, o_ref):
        av0 = a0r[...]
        av1 = a1r[...]
        m = jnp.concatenate([av0[0] + av1[0] + g0r[...],
                             av0[1] + av1[1] + g1r[...]], axis=1)
        dv = deg_ref[...]
        h = m * lax.rsqrt(dv[0] + dv[1] + 1.0) + bg_ref[...]
        t = jnp.maximum(h, 0.0)
        t = jnp.maximum(jnp.dot(t, w1r[...], preferred_element_type=_f32)
                        + s_ref[...] + b1r[...], 0.0)
        t = jnp.maximum(jnp.dot(t, w2r[...], preferred_element_type=_f32)
                        + b2r[...], 0.0)
        o_ref[...] = jnp.maximum(jnp.dot(t, w3r[...],
                                         preferred_element_type=_f32)
                                 + b3r[...], 0.0)
    aspec = pl.BlockSpec((2, _RB, 48), lambda i: (0, i + ioff, 0))
    gspec = pl.BlockSpec((_RB, 48), lambda i: (i + ioff, 0))
    return pl.pallas_call(
        body,
        grid=(_B // _RB,),
        in_specs=[aspec, aspec, gspec, gspec,
                  pl.BlockSpec((2, _RB, 1), lambda i: (0, i + ioff, 0)),
                  pl.BlockSpec((1, _D), lambda i: (0, 0)),
                  pl.BlockSpec((_RB, _F), lambda i: (i + boff, 0)),
                  pl.BlockSpec((_D, _F), lambda i: (0, 0)),
                  pl.BlockSpec((_F, _F), lambda i: (0, 0)),
                  pl.BlockSpec((_F, _D), lambda i: (0, 0)),
                  pl.BlockSpec((1, _F), lambda i: (0, 0)),
                  pl.BlockSpec((1, _F), lambda i: (0, 0)),
                  pl.BlockSpec((1, _D), lambda i: (0, 0))],
        out_specs=pl.BlockSpec((_RB, _D), lambda i: (i, 0)),
        out_shape=jax.ShapeDtypeStruct((_B, _D), _f32),
    )(a0, a1, g0, g1, deg, bg, sr1, w1d, w2, w3, b1, b2, b3)


def _combine_pre(a0, a1, g0, g1, degc, bg, stp, boff, degn, w):
    def body(a0r, a1r, g0r, g1r, dc_ref, bg_ref, stp_ref, dn_ref, w_ref,
             o0, o1):
        av0 = a0r[...]
        av1 = a1r[...]
        m = jnp.concatenate([av0[0] + av1[0] + g0r[...],
                             av0[1] + av1[1] + g1r[...]], axis=1)
        dv = dc_ref[...]
        h = m * lax.rsqrt(dv[0] + dv[1] + 1.0) + bg_ref[...]
        dn = dn_ref[...]
        g = (jnp.dot(h, w_ref[...], preferred_element_type=_f32)
             + stp_ref[...]) * lax.rsqrt(dn[0] + dn[1] + 1.0)
        o0[...] = g[:, 0:48]
        o1[...] = g[:, 48:96]
    aspec = pl.BlockSpec((2, _RB, 48), lambda i: (0, i, 0))
    gspec = pl.BlockSpec((_RB, 48), lambda i: (i, 0))
    return pl.pallas_call(
        body,
        grid=(_B // _RB,),
        in_specs=[aspec, aspec, gspec, gspec,
                  pl.BlockSpec((2, _RB, 1), lambda i: (0, i, 0)),
                  pl.BlockSpec((1, _D), lambda i: (0, 0)),
                  pl.BlockSpec((_RB, _D), lambda i: (i + boff, 0)),
                  pl.BlockSpec((2, _RB, 1), lambda i: (0, i, 0)),
                  pl.BlockSpec((_D, _D), lambda i: (0, 0))],
        out_specs=[pl.BlockSpec((_RB, 48), lambda i: (i, 0))] * 2,
        out_shape=[jax.ShapeDtypeStruct((2 * _B, 48), _f32)] * 2,
    )(a0, a1, g0, g1, degc, bg, stp, degn, w)


def _pre_big(x, wup, deg):
    def body(x_ref, w_ref, deg_ref, *outs):
        dv = deg_ref[...]
        dinv = lax.rsqrt(dv[0] + dv[1] + 1.0)
        g = jnp.dot(x_ref[...], w_ref[...], preferred_element_type=_f32) * dinv
        for k in range(6):
            outs[k][...] = g[:, 16 * k:16 * (k + 1)]
    return pl.pallas_call(
        body,
        grid=(_N // _RB,),
        in_specs=[pl.BlockSpec((_RB, _S), lambda i: (i, 0)),
                  pl.BlockSpec((_S, _D), lambda i: (0, 0)),
                  pl.BlockSpec((2, _RB, 1), lambda i: (0, i, 0))],
        out_specs=[pl.BlockSpec((_RB, 16), lambda i: (i, 0))] * 6,
        out_shape=[jax.ShapeDtypeStruct((_N, 16), _f32)] * 6,
    )(x, wup, deg)


def _combine_big(a0, a1, gs, deg, b):
    def body(a0r, a1r, *rest):
        grs = rest[:6]
        deg_ref, b_ref, o_ref = rest[6:]
        av0 = a0r[...]
        av1 = a1r[...]
        m = jnp.concatenate(
            [av0[k] + av1[k] + grs[k][...] for k in range(6)], axis=1)
        dv = deg_ref[...]
        o_ref[...] = m * lax.rsqrt(dv[0] + dv[1] + 1.0) + b_ref[...]
    aspec = pl.BlockSpec((6, _RB, 16), lambda i: (0, i, 0))
    cspec = pl.BlockSpec((_RB, 16), lambda i: (i, 0))
    return pl.pallas_call(
        body,
        grid=(_N // _RB,),
        in_specs=[aspec, aspec] + [cspec] * 6
                 + [pl.BlockSpec((2, _RB, 1), lambda i: (0, i, 0)),
                    pl.BlockSpec((1, _D), lambda i: (0, 0))],
        out_specs=pl.BlockSpec((_RB, _D), lambda i: (i, 0)),
        out_shape=jax.ShapeDtypeStruct((_N, _D), _f32),
    )(a0, a1, *gs, deg, b)


def _dnn(h, sr1, boff, w1d, w2, w3, b1, b2, b3):
    def body(h_ref, s_ref, w1r, w2r, w3r, b1r, b2r, b3r, o_ref):
        t = jnp.maximum(h_ref[...], 0.0)
        t = jnp.maximum(jnp.dot(t, w1r[...], preferred_element_type=_f32)
                        + s_ref[...] + b1r[...], 0.0)
        t = jnp.maximum(jnp.dot(t, w2r[...], preferred_element_type=_f32)
                        + b2r[...], 0.0)
        o_ref[...] = jnp.maximum(jnp.dot(t, w3r[...],
                                         preferred_element_type=_f32)
                                 + b3r[...], 0.0)
    return pl.pallas_call(
        body,
        grid=(_B // _RB,),
        in_specs=[pl.BlockSpec((_RB, _D), lambda i: (i, 0)),
                  pl.BlockSpec((_RB, _F), lambda i: (i + boff, 0)),
                  pl.BlockSpec((_D, _F), lambda i: (0, 0)),
                  pl.BlockSpec((_F, _F), lambda i: (0, 0)),
                  pl.BlockSpec((_F, _D), lambda i: (0, 0)),
                  pl.BlockSpec((1, _F), lambda i: (0, 0)),
                  pl.BlockSpec((1, _F), lambda i: (0, 0)),
                  pl.BlockSpec((1, _D), lambda i: (0, 0))],
        out_specs=pl.BlockSpec((_RB, _D), lambda i: (i, 0)),
        out_shape=jax.ShapeDtypeStruct((_B, _D), _f32),
    )(h, sr1, w1d, w2, w3, b1, b2, b3)


def _pool(h, ids, wend, bend):
    def body(h_ref, id_ref, we_ref, be_ref, o_ref, acc_ref):
        i = pl.program_id(0)
        @pl.when(i == 0)
        def _():
            acc_ref[...] = jnp.zeros_like(acc_ref)
        oh = (id_ref[...] == lax.broadcasted_iota(jnp.int32, (_RB, _NG), 1)
              ).astype(_f32)
        acc_ref[...] += lax.dot_general(oh, h_ref[...],
                                        (((0,), (0,)), ((), ())),
                                        preferred_element_type=_f32)
        @pl.when(i == pl.num_programs(0) - 1)
        def _():
            o_ref[...] = jnp.maximum(
                jnp.dot(acc_ref[...], we_ref[...],
                        preferred_element_type=_f32) + be_ref[...], 0.0)
    return pl.pallas_call(
        body,
        grid=(_N // _RB,),
        in_specs=[pl.BlockSpec((_RB, _D), lambda i: (i, 0)),
                  pl.BlockSpec((_RB, 1), lambda i: (i, 0)),
                  pl.BlockSpec((_D, 1), lambda i: (0, 0)),
                  pl.BlockSpec((1, 1), lambda i: (0, 0))],
        out_specs=pl.BlockSpec((_NG, 1), lambda i: (0, 0)),
        out_shape=jax.ShapeDtypeStruct((_NG, 1), _f32),
        scratch_shapes=[pltpu.VMEM((_NG, _D), _f32)],
    )(h, ids, wend, bend)


def _sc_mesh():
    return plsc.VectorSubcoreMesh(core_axis_name="c", subcore_axis_name="s",
                                  num_cores=2, num_subcores=16)


@functools.lru_cache(maxsize=None)
def _mk_edgepass(n, ch, w, rows):
    npad = -(-(n + 16) // 128) * 128
    rps = npad // 16
    nread = (n // 16 // 8) * 8
    ntail = n - 16 * nread
    zchunks = []
    off = 0
    while off < rps:
        sz = min(128, rps - off)
        zchunks.append((off, sz))
        off += sz

    k = 4
    ngroups = rows // k
    assert rows % (2 * k) == 0

    def body(*refs):
        gs = refs[:ch]
        srcr, dstr, zr = refs[ch:ch + 3]
        outs = refs[ch + 3:ch + 5]
        sbuf, dbuf, rbufs, zbuf = refs[ch + 5:ch + 9]
        acc = refs[ch + 9]
        semg = refs[ch + 10:ch + 12]
        sems = refs[ch + 12:ch + 14]
        semz = refs[ch + 14]
        c = lax.axis_index("c")
        s = lax.axis_index("s")
        wid = c * 16 + s
        pltpu.sync_copy(srcr.at[pl.ds(wid * rows, rows)], sbuf)
        pltpu.sync_copy(dstr.at[pl.ds(wid * rows, rows)], dbuf)
        pltpu.sync_copy(zr, zbuf)

        def issue_gathers(gref, group, si):
            for b in range(k):
                pltpu.async_copy(gref.at[sbuf.at[group * k + b]],
                                 rbufs.at[si, b], semg[si])

        def drain_gathers(si):
            for b in range(k):
                pltpu.make_async_copy(zr, rbufs.at[si, b], semg[si]).wait()

        for chi in range(ch):
            zds = [pltpu.async_copy(zbuf.at[pl.ds(0, zsz)],
                                    acc.at[pl.ds(s * rps + zo, zsz)], semz)
                   for zo, zsz in zchunks]
            for d in zds:
                d.wait()
            plsc.subcore_barrier()
            issue_gathers(gs[chi], 0, 0)
            issue_gathers(gs[chi], 1, 1)

            def pair(p, carry, chi=chi):
                for si in (0, 1):
                    g = 2 * p + si
                    drain_gathers(si)
                    sds = [pltpu.async_copy(
                        rbufs.at[si, b], acc.at[dbuf.at[g * k + b]],
                        sems[si], add=True) for b in range(k)]
                    for d in sds:
                        d.wait()

                    @pl.when(g + 2 < ngroups)
                    def _(si=si, g=g):
                        issue_gathers(gs[chi], g + 2, si)
                return carry
            lax.fori_loop(0, ngroups // 2, pair, 0)
            plsc.subcore_barrier()
            for ci in range(2):
                @pl.when(c == ci)
                def _(chi=chi, ci=ci):
                    pltpu.sync_copy(acc.at[pl.ds(s * nread, nread)],
                                    outs[ci].at[chi, pl.ds(s * nread,
                                                           nread)])
                    if ntail:
                        @pl.when(s == 0)
                        def _():
                            pltpu.sync_copy(
                                acc.at[pl.ds(16 * nread, ntail)],
                                outs[ci].at[chi, pl.ds(16 * nread, ntail)])
            plsc.subcore_barrier()

    return pl.kernel(
        body,
        out_type=(jax.ShapeDtypeStruct((ch, n, w), _f32),
                  jax.ShapeDtypeStruct((ch, n, w), _f32)),
        mesh=_sc_mesh(),
        compiler_params=pltpu.CompilerParams(use_tc_tiling_on_sc=False),
        scratch_types=[pltpu.VMEM((rows, 128), jnp.int32),
                       pltpu.VMEM((rows, 128), jnp.int32),
                       pltpu.VMEM((2, k, 128, w), _f32),
                       pltpu.VMEM((128, w), _f32),
                       pltpu.VMEM_SHARED((npad, w), _f32),
                       pltpu.SemaphoreType.DMA,
                       pltpu.SemaphoreType.DMA,
                       pltpu.SemaphoreType.DMA,
                       pltpu.SemaphoreType.DMA,
                       pltpu.SemaphoreType.DMA])


@functools.lru_cache(maxsize=None)
def _mk_degree(slots, rows):
    sps = slots // 16
    k = 10
    ngroups = rows // k
    assert rows % (2 * k) == 0

    def body(dstr, zeros_r, out0, out1, dbuf, obuf, stage, acc, sem0, sem1):
        c = lax.axis_index("c")
        s = lax.axis_index("s")
        wid = c * 16 + s
        sems = (sem0, sem1)
        pltpu.sync_copy(dstr.at[pl.ds(wid * rows, rows)], dbuf)
        for i in range(8):
            obuf[pl.ds(i * 16, 16)] = jnp.ones((16,), _f32)
        pltpu.sync_copy(zeros_r, stage)
        pltpu.sync_copy(stage, acc.at[pl.ds(s * sps, sps)])
        plsc.subcore_barrier()

        def issue(group, si):
            for b in range(k):
                pltpu.async_copy(obuf, acc.at[dbuf.at[group * k + b]],
                                 sems[si], add=True)

        def drain(si):
            for b in range(k):
                pltpu.make_async_copy(zeros_r.at[pl.ds(0, 128)], obuf,
                                      sems[si]).wait()
        issue(0, 0)

        def pair(p, carry):
            issue(2 * p + 1, 1)
            drain(0)

            @pl.when(2 * p + 2 < ngroups)
            def _():
                issue(2 * p + 2, 0)
            drain(1)
            return carry
        lax.fori_loop(0, ngroups // 2, pair, 0)
        plsc.subcore_barrier()
        pltpu.sync_copy(acc.at[pl.ds(s * sps, sps)], stage)
        for ci, outr in enumerate((out0, out1)):
            @pl.when(c == ci)
            def _(outr=outr):
                pltpu.sync_copy(stage, outr.at[pl.ds(s * sps, sps)])

    return pl.kernel(
        body,
        out_type=(jax.ShapeDtypeStruct((slots,), _f32),
                  jax.ShapeDtypeStruct((slots,), _f32)),
        mesh=_sc_mesh(),
        compiler_params=pltpu.CompilerParams(use_tc_tiling_on_sc=False),
        scratch_types=[pltpu.VMEM((rows, 128), jnp.int32),
                       pltpu.VMEM((128,), _f32),
                       pltpu.VMEM((sps,), _f32),
                       pltpu.VMEM_SHARED((slots,), _f32),
                       pltpu.SemaphoreType.DMA,
                       pltpu.SemaphoreType.DMA])


def _pad_edges(src, dst, n, epad):
    e = src.shape[0]
    srcp = jnp.concatenate([src, jnp.zeros((epad - e,), jnp.int32)])
    dstp = jnp.concatenate([dst, jnp.full((epad - e,), n, jnp.int32)])
    return srcp.reshape(epad // 128, 128), dstp.reshape(epad // 128, 128), dstp


def kernel(x, feature_mtx_static, edge_index, inner_edges, forward_edges,
           backward_edges, batch_ids, W_up, b_up, W_in, b_in, W_f, b_f,
           W_b, b_b, W1, b1, W2, b2, W3, b3, W_end, b_end):
    st = feature_mtx_static

    stp_i, stp_f, stp_b, sr1 = _stpre(st, W_in[_D:], W_f[_D:], W_b[_D:],
                                      W1[_D:])
    w_in_d, w_f_d, w_b_d, w1_d = W_in[:_D], W_f[:_D], W_b[:_D], W1[:_D]
    b_in2 = b_in.reshape(1, _D)
    b_f2 = b_f.reshape(1, _D)
    b_b2 = b_b.reshape(1, _D)
    b_up2 = b_up.reshape(1, _D)

    biglist = (edge_index[0], edge_index[1], _N, _EP_BIG, _N)
    inn = [(inner_edges[l, 0], inner_edges[l, 1], _B, _EP_SMALL, _B)
           for l in range(_NL)]
    fwd = [(forward_edges[l, 0], forward_edges[l, 1], 2 * _B, _EP_SMALL,
            2 * _B) for l in range(_NL - 1)]
    bwd = [(backward_edges[l, 0], backward_edges[l, 1], 2 * _B, _EP_SMALL,
            2 * _B) for l in range(1, _NL)]

    def prep(lst):
        return [_pad_edges(s_, d_, pd_, ep_)
                for (s_, d_, _n, ep_, pd_) in lst]

    bigP = prep([biglist])[0]
    innP = prep(inn)
    fwdP = prep(fwd)
    bwdP = prep(bwd)

    slots = 140160
    zeros_deg = jnp.zeros((slots // 16,), _f32)
    degfn = _mk_degree(slots, 10240 // 32)
    offsA = [0, 50016, 60032]
    dstA = jnp.concatenate(
        [bigP[2], innP[0][2] + offsA[1], innP[1][2] + offsA[2],
         jnp.full((1310720 - 1146880,), _N, jnp.int32)]).reshape(-1, 128)
    offsB = [0, 10016, 20032, 40048, 60064, 80080, 100096, 120112]
    dstB = jnp.concatenate(
        [innP[2][2], innP[3][2] + offsB[1],
         fwdP[0][2] + offsB[2], fwdP[1][2] + offsB[3],
         fwdP[2][2] + offsB[4],
         bwdP[0][2] + offsB[5], bwdP[1][2] + offsB[6],
         bwdP[2][2] + offsB[7]]).reshape(-1, 128)
    degA = jnp.stack(degfn(dstA, zeros_deg))
    degB = jnp.stack(degfn(dstB, zeros_deg))

    def deg_of(src, off, n_):
        return lax.slice(src, (0, off), (2, off + n_)).reshape(2, n_, 1)

    deg_big = deg_of(degA, 0, _N)
    deg_in = [deg_of(degA, offsA[1], _B), deg_of(degA, offsA[2], _B),
              deg_of(degB, offsB[0], _B), deg_of(degB, offsB[1], _B)]
    deg_fw = [deg_of(degB, offsB[2 + l], 2 * _B) for l in range(_NL - 1)]
    deg_bw = {l: deg_of(degB, offsB[5 + l - 1], 2 * _B)
              for l in range(1, _NL)}

    z48 = jnp.zeros((128, 48), _f32)
    z16 = jnp.zeros((128, 16), _f32)
    ep2 = _mk_edgepass(2 * _B, 2, 48, _EP_SMALL // 128 // 32)
    epbig = _mk_edgepass(_N, 6, 16, _EP_BIG // 128 // 32)

    g6 = _pre_big(x, W_up, deg_big)
    pa0, pa1 = epbig(*g6, bigP[0], bigP[1], z16)
    h_full = _combine_big(pa0, pa1, g6, deg_big, b_up2)
    hb = [lax.slice(h_full, (k * _B, 0), ((k + 1) * _B, _D))
          for k in range(_NL + 1)]

    def gcn_inner(l):
        g0, g1 = _pre(hb[l], stp_i, l * (_B // _RB), deg_in[l], w_in_d)
        a0, a1 = ep2(g0, g1, innP[l][0], innP[l][1], z48)
        hb[l] = _combine(a0, a1, g0, g1, deg_in[l], b_in2, 0, _B)

    for _ in range(_NP):
        for l in range(_NL):
            gcn_inner(l)
            if l == _NL - 1:
                continue
            hcat = jnp.concatenate([hb[l], hb[l + 1]])
            g0, g1 = _pre(hcat, stp_f, l * (_B // _RB), deg_fw[l], w_f_d)
            a0, a1 = ep2(g0, g1, fwdP[l][0], fwdP[l][1], z48)
            hb[l + 1] = _combine(a0, a1, g0, g1, deg_fw[l], b_f2,
                                 _B // _RB, _B)
            hb[l + 1] = _dnn(hb[l + 1], sr1, (l + 1) * (_B // _RB),
                             w1_d, W2, W3, b1.reshape(1, _F),
                             b2.reshape(1, _F), b3.reshape(1, _D))
        for l in range(_NL - 1, 0, -1):
            hcat = jnp.concatenate([hb[l - 1], hb[l]])
            g0, g1 = _pre(hcat, stp_b, (l - 1) * (_B // _RB), deg_bw[l],
                          w_b_d)
            a0, a1 = ep2(g0, g1, bwdP[l - 1][0], bwdP[l - 1][1], z48)
            hb[l - 1] = _combine(a0, a1, g0, g1, deg_bw[l], b_b2, 0, _B)
            gcn_inner(l - 1)
            hb[l - 1] = _dnn(hb[l - 1], sr1, (l - 1) * (_B // _RB),
                             w1_d, W2, W3, b1.reshape(1, _F),
                             b2.reshape(1, _F), b3.reshape(1, _D))

    h_all = jnp.concatenate(hb)
    return _pool(h_all, batch_ids.reshape(_N, 1), W_end, b_end.reshape(1, 1))
